# baseline (speedup 1.0000x reference)
import jax
import jax.numpy as jnp
from jax import lax
from jax.experimental import pallas as pl
from jax.experimental.pallas import tpu as pltpu
from jax.experimental.pallas import tpu_sc as plsc

INPUT_WIDTH = 1024
LEAF_WIDTH = 16
OUTPUT_WIDTH = 1024
DEPTH = 11
N_LEAVES = 2 ** DEPTH
N_NODES = 2 ** DEPTH - 1
BATCH = 8
LANES = 16
NEG_INF = -3.0e38
N_CHUNKS = INPUT_WIDTH // LANES
UNROLL = 8


def _fff_body(x_hbm, nw_hbm, nb_hbm, w1_hbm, b1_hbm, w2_hbm, b2_hbm, out_hbm,
              x_v, nb_v, b1_v, w1_v, w2_v, b2_v, y_v, e_v,
              blk0, blk1, blk2, blk3,
              sem0, sem1, sem2, semb0, semb1, semb2, semb3):
    wid = lax.axis_index("s")
    blks = [blk0, blk1, blk2, blk3]
    bsems = [semb0, semb1, semb2, semb3]

    def slot(d):
        return 3 if d <= 1 else (d - 2) % 3

    @pl.when((lax.axis_index("c") == 0) & (wid < BATCH))
    def _():
        b = wid
        cx = pltpu.async_copy(x_hbm.at[b], x_v, sem0)
        cnb = pltpu.async_copy(nb_hbm, nb_v, sem1)
        cb1 = pltpu.async_copy(b1_hbm, b1_v, sem2)
        cx.wait()
        cnb.wait()

        def fetch_block(base, sl):
            return [pltpu.async_copy(nw_hbm.at[base + i],
                                     blks[sl].at[pl.ds(i * INPUT_WIDTH,
                                                       INPUT_WIDTH)],
                                     bsems[sl])
                    for i in range(4)]

        copies = {}
        copies[0] = fetch_block(jnp.int32(0), slot(0))
        copies[1] = copies[0]

        prefix = jnp.int32(0)
        decs = []
        for d in range(DEPTH):
            if d + 2 < DEPTH:
                nxt = d + 2
                base = jnp.int32(2 ** nxt - 1) + 4 * prefix
                copies[nxt] = fetch_block(base, slot(nxt))
            if d != 1:
                for c in copies[d]:
                    c.wait()
            blk = blks[slot(d)]
            if d == 0:
                off = jnp.int32(0)
            elif d == 1:
                off = 1 + decs[0]
            else:
                off = 2 * decs[d - 2] + decs[d - 1]
            boff = off * INPUT_WIDTH

            def dot_chunk(k, acc, _blk=blk, _boff=boff):
                o = pl.multiple_of(k * (LANES * UNROLL), LANES)
                for u in range(UNROLL):
                    ou = o + u * LANES
                    acc = acc + (x_v[pl.ds(ou, LANES)]
                                 * _blk[pl.ds(pl.multiple_of(_boff + ou,
                                                             LANES), LANES)])
                return acc

            acc = lax.fori_loop(0, N_CHUNKS // UNROLL, dot_chunk,
                                jnp.zeros((LANES,), jnp.float32))
            node = jnp.int32(2 ** d - 1) + prefix
            bias = plsc.load_gather(nb_v, [jnp.full((LANES,), node, jnp.int32)])
            logit = jnp.sum(acc) + bias[0]
            dec = (logit > 0.0).astype(jnp.int32)
            prefix = 2 * prefix + dec
            decs.append(dec)

        leaf = prefix

        c1 = pltpu.async_copy(w1_hbm.at[leaf], w1_v, sem0)
        c2 = pltpu.async_copy(w2_hbm.at[leaf], w2_v, sem1)
        cb1.wait()
        c1.wait()

        def hstep(c, accs):
            o = pl.multiple_of(c * (2 * LANES), LANES)
            out = list(accs)
            for u in range(2):
                ou = o + u * LANES
                xc = x_v[pl.ds(ou, LANES)]
                for j in range(LEAF_WIDTH):
                    out[j] = out[j] + xc * w1_v[j, pl.ds(ou, LANES)]
            return tuple(out)

        accs = lax.fori_loop(
            0, N_CHUNKS // 2, hstep,
            tuple(jnp.zeros((LANES,), jnp.float32) for _ in range(LEAF_WIDTH)))
        b1g = plsc.load_gather(
            b1_v, [lax.iota(jnp.int32, LANES),
                   jnp.full((LANES,), leaf, jnp.int32)])
        hs = []
        for j in range(LEAF_WIDTH):
            hj = jnp.sum(accs[j]) + b1g[j]
            hs.append(jnp.where(hj > 0.0, hj, 0.0))

        cb2 = pltpu.async_copy(b2_hbm.at[leaf], b2_v, sem2)
        c2.wait()
        cb2.wait()

        def ystep(c, mv):
            o = pl.multiple_of(c * (2 * LANES), LANES)
            for u in range(2):
                ou = o + u * LANES
                yv = b2_v[pl.ds(ou, LANES)]
                for j in range(LEAF_WIDTH):
                    yv = yv + w2_v[j, pl.ds(ou, LANES)] * hs[j]
                y_v[pl.ds(ou, LANES)] = yv
                mv = jnp.maximum(mv, yv)
            return mv

        mv = lax.fori_loop(0, N_CHUNKS // 2, ystep,
                           jnp.full((LANES,), NEG_INF, jnp.float32))
        m = jnp.max(mv)

        def estep(c, sv):
            o = pl.multiple_of(c * (LANES * UNROLL), LANES)
            for u in range(UNROLL):
                ou = o + u * LANES
                ev = jnp.exp(y_v[pl.ds(ou, LANES)] - m)
                e_v[pl.ds(ou, LANES)] = ev
                sv = sv + ev
            return sv

        sv = lax.fori_loop(0, N_CHUNKS // UNROLL, estep,
                           jnp.zeros((LANES,), jnp.float32))
        s = jnp.sum(sv)
        inv_v = jnp.ones((LANES,), jnp.float32) / jnp.full((LANES,), s)

        def nstep(c, carry):
            o = pl.multiple_of(c * (LANES * UNROLL), LANES)
            for u in range(UNROLL):
                ou = o + u * LANES
                e_v[pl.ds(ou, LANES)] = e_v[pl.ds(ou, LANES)] * inv_v
            return carry

        lax.fori_loop(0, N_CHUNKS // UNROLL, nstep, jnp.int32(0))
        pltpu.sync_copy(e_v, out_hbm.at[b])


def kernel(x, node_weights, node_biases, w1s, b1s, w2s, b2s):
    w1t = jnp.transpose(w1s, (0, 2, 1))
    b1t = jnp.transpose(b1s, (1, 0))
    nb = jnp.reshape(node_biases, (N_NODES,))
    mesh = plsc.VectorSubcoreMesh(core_axis_name="c", subcore_axis_name="s",
                                  num_cores=1)
    f = pl.kernel(
        _fff_body,
        out_type=jax.ShapeDtypeStruct((BATCH, OUTPUT_WIDTH), jnp.float32),
        mesh=mesh,
        compiler_params=pltpu.CompilerParams(
            needs_layout_passes=False, use_tc_tiling_on_sc=True),
        scratch_types=[
            pltpu.VMEM((INPUT_WIDTH,), jnp.float32),
            pltpu.VMEM((N_NODES,), jnp.float32),
            pltpu.VMEM((LEAF_WIDTH, N_LEAVES), jnp.float32),
            pltpu.VMEM((LEAF_WIDTH, INPUT_WIDTH), jnp.float32),
            pltpu.VMEM((LEAF_WIDTH, OUTPUT_WIDTH), jnp.float32),
            pltpu.VMEM((OUTPUT_WIDTH,), jnp.float32),
            pltpu.VMEM((OUTPUT_WIDTH,), jnp.float32),
            pltpu.VMEM((OUTPUT_WIDTH,), jnp.float32),
            pltpu.VMEM((4 * INPUT_WIDTH,), jnp.float32),
            pltpu.VMEM((4 * INPUT_WIDTH,), jnp.float32),
            pltpu.VMEM((4 * INPUT_WIDTH,), jnp.float32),
            pltpu.VMEM((4 * INPUT_WIDTH,), jnp.float32),
            pltpu.SemaphoreType.DMA,
            pltpu.SemaphoreType.DMA,
            pltpu.SemaphoreType.DMA,
            pltpu.SemaphoreType.DMA,
            pltpu.SemaphoreType.DMA,
            pltpu.SemaphoreType.DMA,
            pltpu.SemaphoreType.DMA,
        ],
    )
    return f(x, node_weights, nb, w1t, b1t, w2s, b2s)
